# Initial kernel scaffold; baseline (speedup 1.0000x reference)
import functools

import jax
import jax.numpy as jnp
from jax import lax
from jax.experimental import pallas as pl
from jax.experimental.pallas import tpu as pltpu
from jax.experimental.pallas import tpu_sc as plsc

N = 10000
E = 320000
D = 128
H = 128

NC = 2
NS = 16
L = 16
NW = NC * NS

CH = 128
KCH = 82
EP = NW * KCH * CH
PADE = EP - (E + N)
NPAD = 10240
RPT = NPAD // NS

_mesh = plsc.VectorSubcoreMesh(core_axis_name="c", subcore_axis_name="s")


@functools.partial(
    pl.kernel,
    mesh=_mesh,
    out_type=jax.ShapeDtypeStruct((NC, NPAD), jnp.float32),
    scratch_types=[
        pltpu.VMEM((KCH, CH), jnp.int32),
        pltpu.VMEM((CH,), jnp.float32),
        pltpu.VMEM_SHARED((NPAD,), jnp.float32),
        pltpu.SemaphoreType.DMA,
    ],
)
def _deg_kernel(cols_hbm, zeros1_hbm, out_hbm, colv, ones_v, acc, sem):
    c = lax.axis_index("c")
    s = lax.axis_index("s")
    wid = s * NC + c
    pltpu.sync_copy(zeros1_hbm.at[pl.ds(s * RPT, RPT)],
                    acc.at[pl.ds(s * RPT, RPT)])
    for i in range(CH // L):
        ones_v[pl.ds(i * L, L)] = jnp.ones((L,), jnp.float32)
    pltpu.sync_copy(cols_hbm.at[pl.ds(wid * KCH, KCH)], colv)
    plsc.subcore_barrier()

    def body(j, _):
        pltpu.sync_copy(ones_v, acc.at[colv.at[j]], add=True)
        return 0

    lax.fori_loop(0, KCH, body, 0)
    plsc.subcore_barrier()
    pltpu.sync_copy(acc.at[pl.ds(s * RPT, RPT)],
                    out_hbm.at[c, pl.ds(s * RPT, RPT)])


@functools.partial(
    pl.kernel,
    mesh=_mesh,
    out_type=jax.ShapeDtypeStruct((NC, NPAD, H), jnp.float32),
    scratch_types=[
        pltpu.VMEM((KCH, CH), jnp.int32),
        pltpu.VMEM((KCH, CH), jnp.int32),
        pltpu.VMEM((CH, H), jnp.float32),
        pltpu.VMEM_SHARED((NPAD, H), jnp.float32),
        pltpu.SemaphoreType.DMA,
    ],
)
def _agg_kernel(y_hbm, rows_hbm, cols_hbm, zeros2_hbm, out_hbm,
                rowv, colv, gbuf, acc, sem):
    c = lax.axis_index("c")
    s = lax.axis_index("s")
    wid = s * NC + c
    pltpu.sync_copy(zeros2_hbm.at[pl.ds(s * RPT, RPT)],
                    acc.at[pl.ds(s * RPT, RPT)])
    pltpu.sync_copy(rows_hbm.at[pl.ds(wid * KCH, KCH)], rowv)
    pltpu.sync_copy(cols_hbm.at[pl.ds(wid * KCH, KCH)], colv)
    plsc.subcore_barrier()

    def body(j, _):
        pltpu.async_copy(y_hbm.at[rowv.at[j]], gbuf, sem).wait()
        pltpu.sync_copy(gbuf, acc.at[colv.at[j]], add=True)
        return 0

    lax.fori_loop(0, KCH, body, 0)
    plsc.subcore_barrier()
    pltpu.sync_copy(acc.at[pl.ds(s * RPT, RPT)],
                    out_hbm.at[c, pl.ds(s * RPT, RPT)])


def _mm_scale_body(x_ref, w_ref, d0_ref, d1_ref, y_ref, dinv_ref):
    dinv = lax.rsqrt(d0_ref[...] + d1_ref[...])
    y_ref[...] = jnp.dot(x_ref[...], w_ref[...],
                         preferred_element_type=jnp.float32) * dinv
    dinv_ref[...] = dinv


_mm_scale = pl.pallas_call(
    _mm_scale_body,
    out_shape=[jax.ShapeDtypeStruct((N, H), jnp.float32),
               jax.ShapeDtypeStruct((N, 1), jnp.float32)],
)


def _mid_body(p0_ref, p1_ref, dinv_ref, b_ref, w_ref, y_ref):
    dinv = dinv_ref[...]
    h = jnp.maximum((p0_ref[...] + p1_ref[...]) * dinv + b_ref[...], 0.0)
    y_ref[...] = jnp.dot(h, w_ref[...],
                         preferred_element_type=jnp.float32) * dinv


_mid = pl.pallas_call(
    _mid_body,
    out_shape=jax.ShapeDtypeStruct((N, H), jnp.float32),
)


def _final_body(p0_ref, p1_ref, dinv_ref, b_ref, out_ref):
    out_ref[...] = jnp.maximum(
        (p0_ref[...] + p1_ref[...]) * dinv_ref[...] + b_ref[...], 0.0)


_final = pl.pallas_call(
    _final_body,
    out_shape=jax.ShapeDtypeStruct((N, H), jnp.float32),
)


def kernel(x, edge_index, W1, b1, W2, b2):
    loop = jnp.arange(N, dtype=jnp.int32)
    rows = jnp.concatenate(
        [edge_index[0], loop, jnp.zeros((PADE,), jnp.int32)]
    ).reshape(NW * KCH, CH)
    cols = jnp.concatenate(
        [edge_index[1], loop, jnp.full((PADE,), N, jnp.int32)]
    ).reshape(NW * KCH, CH)
    zeros1 = jnp.zeros((NPAD,), jnp.float32)
    zeros2 = jnp.zeros((NPAD, H), jnp.float32)

    degp = _deg_kernel(cols, zeros1)
    d0 = degp[0, :N].reshape(N, 1)
    d1 = degp[1, :N].reshape(N, 1)

    y1, dinv = _mm_scale(x, W1, d0, d1)
    p = _agg_kernel(y1, rows, cols, zeros2)
    y2 = _mid(p[0, :N], p[1, :N], dinv, b1.reshape(1, H), W2)
    q = _agg_kernel(y2, rows, cols, zeros2)
    return _final(q[0, :N], q[1, :N], dinv, b2.reshape(1, H))


# speedup vs baseline: 3.6554x; 3.6554x over previous
import functools

import jax
import jax.numpy as jnp
from jax import lax
from jax.experimental import pallas as pl
from jax.experimental.pallas import tpu as pltpu
from jax.experimental.pallas import tpu_sc as plsc

N = 10000
E = 320000
D = 128
H = 128

NC = 2
NS = 16
L = 16
NW = NC * NS

CH = 128
KCH = 88
EP = NW * KCH * CH
PADE = EP - (E + N)
NPAD = 10240
RPT = NPAD // NS

_mesh = plsc.VectorSubcoreMesh(core_axis_name="c", subcore_axis_name="s")


@functools.partial(
    pl.kernel,
    mesh=_mesh,
    out_type=jax.ShapeDtypeStruct((NC, NPAD), jnp.float32),
    scratch_types=[
        pltpu.VMEM((KCH, CH), jnp.int32),
        pltpu.VMEM((CH,), jnp.float32),
        pltpu.VMEM_SHARED((NPAD,), jnp.float32),
        pltpu.SemaphoreType.DMA,
    ],
)
def _deg_kernel(cols_hbm, zeros1_hbm, out_hbm, colv, ones_v, acc, sem):
    c = lax.axis_index("c")
    s = lax.axis_index("s")
    wid = s * NC + c
    pltpu.sync_copy(zeros1_hbm.at[pl.ds(s * RPT, RPT)],
                    acc.at[pl.ds(s * RPT, RPT)])
    for i in range(CH // L):
        ones_v[pl.ds(i * L, L)] = jnp.ones((L,), jnp.float32)
    pltpu.sync_copy(cols_hbm.at[pl.ds(wid * KCH, KCH)], colv)
    plsc.subcore_barrier()

    def body(j, _):
        pltpu.sync_copy(ones_v, acc.at[colv.at[j]], add=True)
        return 0

    lax.fori_loop(0, KCH, body, 0)
    plsc.subcore_barrier()
    pltpu.sync_copy(acc.at[pl.ds(s * RPT, RPT)],
                    out_hbm.at[c, pl.ds(s * RPT, RPT)])


@functools.partial(
    pl.kernel,
    mesh=_mesh,
    out_type=jax.ShapeDtypeStruct((NC, NPAD, H), jnp.float32),
    scratch_types=[
        pltpu.VMEM((KCH, CH), jnp.int32),
        pltpu.VMEM((KCH, CH), jnp.int32),
        pltpu.VMEM((CH, H), jnp.float32),
        pltpu.VMEM_SHARED((NPAD, H), jnp.float32),
        pltpu.SemaphoreType.DMA,
    ],
)
def _agg_kernel(y_hbm, rows_hbm, cols_hbm, zeros2_hbm, out_hbm,
                rowv, colv, gbuf, acc, sem):
    c = lax.axis_index("c")
    s = lax.axis_index("s")
    wid = s * NC + c
    pltpu.sync_copy(zeros2_hbm.at[pl.ds(s * RPT, RPT)],
                    acc.at[pl.ds(s * RPT, RPT)])
    pltpu.sync_copy(rows_hbm.at[pl.ds(wid * KCH, KCH)], rowv)
    pltpu.sync_copy(cols_hbm.at[pl.ds(wid * KCH, KCH)], colv)
    plsc.subcore_barrier()

    def body(j, _):
        pltpu.async_copy(y_hbm.at[rowv.at[j]], gbuf, sem).wait()
        pltpu.sync_copy(gbuf, acc.at[colv.at[j]], add=True)
        return 0

    lax.fori_loop(0, KCH, body, 0)
    plsc.subcore_barrier()
    pltpu.sync_copy(acc.at[pl.ds(s * RPT, RPT)],
                    out_hbm.at[c, pl.ds(s * RPT, RPT)])


def _mm_scale_body(x_ref, w_ref, d0_ref, d1_ref, y_ref, dinv_ref):
    dinv = lax.rsqrt(d0_ref[...] + d1_ref[...])
    y_ref[...] = jnp.dot(x_ref[...], w_ref[...],
                         preferred_element_type=jnp.float32) * dinv
    dinv_ref[...] = dinv


_mm_scale = pl.pallas_call(
    _mm_scale_body,
    out_shape=[jax.ShapeDtypeStruct((N, H), jnp.float32),
               jax.ShapeDtypeStruct((N, 1), jnp.float32)],
)


def _mid_body(p0_ref, p1_ref, dinv_ref, b_ref, w_ref, y_ref):
    dinv = dinv_ref[...]
    h = jnp.maximum((p0_ref[...] + p1_ref[...]) * dinv + b_ref[...], 0.0)
    y_ref[...] = jnp.dot(h, w_ref[...],
                         preferred_element_type=jnp.float32) * dinv


_mid = pl.pallas_call(
    _mid_body,
    out_shape=jax.ShapeDtypeStruct((N, H), jnp.float32),
)


def _final_body(p0_ref, p1_ref, dinv_ref, b_ref, out_ref):
    out_ref[...] = jnp.maximum(
        (p0_ref[...] + p1_ref[...]) * dinv_ref[...] + b_ref[...], 0.0)


_final = pl.pallas_call(
    _final_body,
    out_shape=jax.ShapeDtypeStruct((N, H), jnp.float32),
)


def kernel(x, edge_index, W1, b1, W2, b2):
    loop = jnp.arange(N, dtype=jnp.int32)
    rows = jnp.concatenate(
        [edge_index[0], loop, jnp.zeros((PADE,), jnp.int32)]
    ).reshape(NW * KCH, CH)
    dummy_cols = N + jnp.arange(PADE, dtype=jnp.int32) % (NPAD - N)
    cols = jnp.concatenate(
        [edge_index[1], loop, dummy_cols]
    ).reshape(NW * KCH, CH)
    zeros1 = jnp.zeros((NPAD,), jnp.float32)
    zeros2 = jnp.zeros((NPAD, H), jnp.float32)

    degp = _deg_kernel(cols, zeros1)
    d0 = degp[0, :N].reshape(N, 1)
    d1 = degp[1, :N].reshape(N, 1)

    y1, dinv = _mm_scale(x, W1, d0, d1)
    p = _agg_kernel(y1, rows, cols, zeros2)
    y2 = _mid(p[0, :N], p[1, :N], dinv, b1.reshape(1, H), W2)
    q = _agg_kernel(y2, rows, cols, zeros2)
    return _final(q[0, :N], q[1, :N], dinv, b2.reshape(1, H))
